# baseline (speedup 1.0000x reference)
import functools

import jax
import jax.numpy as jnp
from jax.experimental import pallas as pl

N_EMBED = 2048
NUM_EXPERTS = 64
TOP_K = 8
T_TOKENS = 16384
BLOCK_R = 1024
W_COLS = 256


def _router_body(x_ref, w_ref, b_ref, eps_ref, router_ref, idx_ref, skip_ref):
    z = jnp.dot(x_ref[:], w_ref[:], preferred_element_type=jnp.float32,
                precision=jax.lax.Precision.HIGHEST)
    z = z + b_ref[:]
    logits = z[:, :NUM_EXPERTS]
    noise_logits = z[:, NUM_EXPERTS:2 * NUM_EXPERTS]
    skip_logit = z[:, 2 * NUM_EXPERTS:2 * NUM_EXPERTS + 1]

    noisy = logits + eps_ref[:] * jax.nn.softplus(noise_logits)

    noisy_t = noisy.T
    rows = noisy.shape[0]
    col = jax.lax.broadcasted_iota(jnp.int32, (NUM_EXPERTS, rows), 0)
    vals = noisy_t
    sel = jnp.zeros((NUM_EXPERTS, rows), jnp.bool_)
    idx_rows = []
    tok_max = None
    for k in range(TOP_K):
        m = jnp.max(vals, axis=0, keepdims=True)
        if k == 0:
            tok_max = m
        is_m = vals == m
        idx = jnp.min(jnp.where(is_m, col, NUM_EXPERTS), axis=0, keepdims=True)
        hit = col == idx
        sel = jnp.logical_or(sel, hit)
        vals = jnp.where(hit, -jnp.inf, vals)
        idx_rows.append(idx)
    idx_ref[:] = jnp.concatenate(idx_rows, axis=0).T

    e = jnp.where(sel, jnp.exp(noisy_t - tok_max), 0.0)
    denom = jnp.sum(e, axis=0, keepdims=True)
    router_ref[:] = (e / denom).T
    skip_ref[:] = jax.nn.sigmoid(skip_logit)


@functools.partial(jax.jit, static_argnames=("interpret",))
def kernel(x, Wr, br, Wn, bn, Ws, bs, eps, interpret=False):
    w = jnp.concatenate(
        [Wr, Wn, Ws, jnp.zeros((W_COLS - 2 * NUM_EXPERTS - 1, N_EMBED), jnp.float32)],
        axis=0,
    ).T
    b = jnp.concatenate(
        [br, bn, bs, jnp.zeros((W_COLS - 2 * NUM_EXPERTS - 1,), jnp.float32)]
    )[None, :]

    grid = (T_TOKENS // BLOCK_R,)
    router, indices, skip = pl.pallas_call(
        _router_body,
        grid=grid,
        in_specs=[
            pl.BlockSpec((BLOCK_R, N_EMBED), lambda i: (i, 0)),
            pl.BlockSpec((N_EMBED, W_COLS), lambda i: (0, 0)),
            pl.BlockSpec((1, W_COLS), lambda i: (0, 0)),
            pl.BlockSpec((BLOCK_R, NUM_EXPERTS), lambda i: (i, 0)),
        ],
        out_specs=[
            pl.BlockSpec((BLOCK_R, NUM_EXPERTS), lambda i: (i, 0)),
            pl.BlockSpec((BLOCK_R, TOP_K), lambda i: (i, 0)),
            pl.BlockSpec((BLOCK_R, 1), lambda i: (i, 0)),
        ],
        out_shape=[
            jax.ShapeDtypeStruct((T_TOKENS, NUM_EXPERTS), jnp.float32),
            jax.ShapeDtypeStruct((T_TOKENS, TOP_K), jnp.int32),
            jax.ShapeDtypeStruct((T_TOKENS, 1), jnp.float32),
        ],
        interpret=interpret,
    )(x, w, b, eps)
    return (router, indices, skip)


# speedup vs baseline: 1.8352x; 1.8352x over previous
import functools

import jax
import jax.numpy as jnp
from jax.experimental import pallas as pl

N_EMBED = 2048
NUM_EXPERTS = 64
TOP_K = 8
T_TOKENS = 16384
BLOCK_R = 2048
W_COLS = 256


def _router_body(x_ref, w_ref, b_ref, eps_ref, router_ref, idx_ref, skip_ref):
    z = jnp.dot(x_ref[:], w_ref[:], preferred_element_type=jnp.float32)
    z = z + b_ref[:]
    logits = z[:, :NUM_EXPERTS]
    noise_logits = z[:, NUM_EXPERTS:2 * NUM_EXPERTS]
    skip_logit = z[:, 2 * NUM_EXPERTS:2 * NUM_EXPERTS + 1]

    noisy = logits + eps_ref[:] * jax.nn.softplus(noise_logits)

    noisy_t = noisy.T
    rows = noisy.shape[0]
    col = jax.lax.broadcasted_iota(jnp.int32, (NUM_EXPERTS, rows), 0)
    vals = noisy_t
    sel = jnp.zeros((NUM_EXPERTS, rows), jnp.bool_)
    idx_rows = []
    tok_max = None
    for k in range(TOP_K):
        m = jnp.max(vals, axis=0, keepdims=True)
        if k == 0:
            tok_max = m
        is_m = vals == m
        idx = jnp.min(jnp.where(is_m, col, NUM_EXPERTS), axis=0, keepdims=True)
        hit = col == idx
        sel = jnp.logical_or(sel, hit)
        vals = jnp.where(hit, -jnp.inf, vals)
        idx_rows.append(idx)
    idx_ref[:] = jnp.concatenate(idx_rows, axis=0).T

    e = jnp.where(sel, jnp.exp(noisy_t - tok_max), 0.0)
    denom = jnp.sum(e, axis=0, keepdims=True)
    router_ref[:] = (e / denom).T
    skip_ref[:] = jax.nn.sigmoid(skip_logit)


@functools.partial(jax.jit, static_argnames=("interpret",))
def kernel(x, Wr, br, Wn, bn, Ws, bs, eps, interpret=False):
    w = jnp.concatenate(
        [Wr, Wn, Ws, jnp.zeros((W_COLS - 2 * NUM_EXPERTS - 1, N_EMBED), jnp.float32)],
        axis=0,
    ).T
    b = jnp.concatenate(
        [br, bn, bs, jnp.zeros((W_COLS - 2 * NUM_EXPERTS - 1,), jnp.float32)]
    )[None, :]

    grid = (T_TOKENS // BLOCK_R,)
    router, indices, skip = pl.pallas_call(
        _router_body,
        grid=grid,
        in_specs=[
            pl.BlockSpec((BLOCK_R, N_EMBED), lambda i: (i, 0)),
            pl.BlockSpec((N_EMBED, W_COLS), lambda i: (0, 0)),
            pl.BlockSpec((1, W_COLS), lambda i: (0, 0)),
            pl.BlockSpec((BLOCK_R, NUM_EXPERTS), lambda i: (i, 0)),
        ],
        out_specs=[
            pl.BlockSpec((BLOCK_R, NUM_EXPERTS), lambda i: (i, 0)),
            pl.BlockSpec((BLOCK_R, TOP_K), lambda i: (i, 0)),
            pl.BlockSpec((BLOCK_R, 1), lambda i: (i, 0)),
        ],
        out_shape=[
            jax.ShapeDtypeStruct((T_TOKENS, NUM_EXPERTS), jnp.float32),
            jax.ShapeDtypeStruct((T_TOKENS, TOP_K), jnp.int32),
            jax.ShapeDtypeStruct((T_TOKENS, 1), jnp.float32),
        ],
        interpret=interpret,
    )(x, w, b, eps)
    return (router, indices, skip)
